# baseline (device time: 9375 ns/iter reference)
import jax
import jax.numpy as jnp
from jax import lax
from jax.experimental import pallas as pl
from jax.experimental.pallas import tpu as pltpu

CH = 64


def kernel(x, dest):
    n, d = x.shape
    nc_max = n // CH

    def body(x_ref, dest_ref, out_ref, send_buf, staging, send_sems,
             recv_sems):
        my_x = lax.axis_index("x")
        my_y = lax.axis_index("y")
        my_z = lax.axis_index("z")
        peer = (my_x, my_y, 1 - my_z)

        barrier_sem = pltpu.get_barrier_semaphore()
        pl.semaphore_signal(
            barrier_sem, inc=1, device_id=peer,
            device_id_type=pl.DeviceIdType.MESH,
        )

        dst = dest_ref[:]
        keep = (dst == my_z).astype(jnp.int32)
        col = lax.broadcasted_iota(jnp.int32, (1, n), 1)
        ks = keep
        s = 1
        while s < n:
            ks = ks + jnp.where(col >= s, pltpu.roll(ks, s, 1), 0)
            s *= 2
        kb = ks - keep
        sb = col - kb

        c_keep = jnp.sum(keep)
        c_send = n - c_keep
        off_keep = my_z * c_send
        off_xfer = my_z * c_keep
        off_recv = (1 - my_z) * c_keep
        xfer_end = off_xfer + c_send

        x_b = x_ref[:].astype(jnp.bfloat16)

        t_send = off_xfer + sb - keep * 2048

        def chunk_rdma(k):
            return pltpu.make_async_remote_copy(
                src_ref=send_buf.at[pl.ds(k * CH, CH)],
                dst_ref=staging.at[pl.ds(k * CH, CH)],
                send_sem=send_sems.at[k],
                recv_sem=recv_sems.at[k],
                device_id=peer,
                device_id_type=pl.DeviceIdType.MESH,
            )

        def live(k):
            return ((k + 1) * CH > off_xfer) & (k * CH < xfer_end)

        def live_recv(k):
            return ((k + 1) * CH > off_recv) & (k * CH < off_recv + c_send)

        for k in range(nc_max):
            @pl.when(live(k))
            def _(k=k):
                ri_t = k * CH + lax.broadcasted_iota(jnp.int32, (CH, n), 0)
                p_t = (ri_t == t_send).astype(jnp.bfloat16)
                send_buf[pl.ds(k * CH, CH), :] = jnp.dot(
                    p_t, x_b, preferred_element_type=jnp.float32
                ).astype(jnp.bfloat16)

        pl.semaphore_wait(barrier_sem, 1)

        for k in range(nc_max):
            @pl.when(live(k))
            def _(k=k):
                chunk_rdma(k).start()

        ri = lax.broadcasted_iota(jnp.int32, (n, n), 0)
        t_keep = off_keep + kb - (1 - keep) * 2048
        p_local = (ri == t_keep).astype(jnp.bfloat16)
        local_part = jnp.dot(
            p_local, x_b, preferred_element_type=jnp.float32
        ).astype(jnp.bfloat16)
        r1 = lax.broadcasted_iota(jnp.int32, (n, 1), 0)
        in_recv = (r1 >= off_recv) & (r1 < off_recv + c_send)

        for k in range(nc_max):
            @pl.when(live_recv(k))
            def _(k=k):
                chunk_rdma(k).wait_recv()

        out_ref[:, :] = jnp.where(in_recv, staging[:, :], local_part)

        for k in range(nc_max):
            @pl.when(live(k))
            def _(k=k):
                chunk_rdma(k).wait_send()

    return pl.pallas_call(
        body,
        out_shape=jax.ShapeDtypeStruct((n, d), jnp.bfloat16),
        in_specs=[
            pl.BlockSpec(memory_space=pltpu.VMEM),
            pl.BlockSpec(memory_space=pltpu.VMEM),
        ],
        out_specs=pl.BlockSpec(memory_space=pltpu.VMEM),
        scratch_shapes=[
            pltpu.VMEM((n, d), jnp.bfloat16),
            pltpu.VMEM((n, d), jnp.bfloat16),
            pltpu.SemaphoreType.DMA((nc_max,)),
            pltpu.SemaphoreType.DMA((nc_max,)),
        ],
        compiler_params=pltpu.CompilerParams(collective_id=0),
    )(x, dest.reshape(1, n))
